# initial kernel scaffold (unmeasured)
import jax
import jax.numpy as jnp
from jax import lax
from jax.experimental import pallas as pl
from jax.experimental.pallas import tpu as pltpu


def kernel(
    x,
):
    def body(*refs):
        pass

    out_shape = jax.ShapeDtypeStruct(..., jnp.float32)
    return pl.pallas_call(body, out_shape=out_shape)(...)


# baseline (device time: 64804 ns/iter reference)
import jax
import jax.numpy as jnp
from jax import lax
from jax.experimental import pallas as pl
from jax.experimental.pallas import tpu as pltpu

N_DEV = 32
N_STEPS = 5

_FLIPS = ((1, 0, 0), (0, 1, 0), (0, 2, 0), (0, 0, 1), (0, 0, 2))


def _coords_from_logical(me):
    z = me // 8
    r = me % 8
    y = r // 2
    x = (r % 2) ^ (y & 1)
    return x, y, z


def _logical_from_coords(x, y, z):
    return z * 8 + y * 2 + (x ^ (y & 1))


def kernel(x):
    m, n = x.shape

    def body(x_ref, out_ref, acc_ref, send_buf, recv_bufs, send_sems, recv_sems):
        me = lax.axis_index("i")
        mx, my, mz = _coords_from_logical(me)

        acc_ref[...] = x_ref[...]

        for k, (dx, dy, dz) in enumerate(_FLIPS):
            partner = _logical_from_coords(mx ^ dx, my ^ dy, mz ^ dz)
            send_buf[...] = acc_ref[...].astype(jnp.bfloat16)
            rdma = pltpu.make_async_remote_copy(
                src_ref=send_buf,
                dst_ref=recv_bufs.at[k],
                send_sem=send_sems.at[k],
                recv_sem=recv_sems.at[k],
                device_id=partner,
                device_id_type=pl.DeviceIdType.LOGICAL,
            )
            rdma.start()
            rdma.wait()
            acc_ref[...] += recv_bufs[k].astype(jnp.float32)

        out_ref[...] = acc_ref[...]

    return pl.pallas_call(
        body,
        out_shape=jax.ShapeDtypeStruct((m, n), jnp.float32),
        in_specs=[pl.BlockSpec(memory_space=pltpu.VMEM)],
        out_specs=pl.BlockSpec(memory_space=pltpu.VMEM),
        scratch_shapes=[
            pltpu.VMEM((m, n), jnp.float32),
            pltpu.VMEM((m, n), jnp.bfloat16),
            pltpu.VMEM((N_STEPS, m, n), jnp.bfloat16),
            pltpu.SemaphoreType.DMA((N_STEPS,)),
            pltpu.SemaphoreType.DMA((N_STEPS,)),
        ],
    )(x)


# device time: 42266 ns/iter; 1.5332x vs baseline; 1.5332x over previous
import jax
import jax.numpy as jnp
from jax import lax
from jax.experimental import pallas as pl
from jax.experimental.pallas import tpu as pltpu

N_DEV = 32
N_DIMS = 5


def kernel(x):
    m, n = x.shape
    half = [m >> (k + 1) for k in range(N_DIMS)]

    def body(x_ref, out_ref, acc_ref, stage_ref, *rest):
        rs_recv = rest[0:N_DIMS]
        ag_recv = rest[N_DIMS : 2 * N_DIMS]
        send_sems, recv_sems = rest[2 * N_DIMS], rest[2 * N_DIMS + 1]

        me = lax.axis_index("i")
        mz = me // 8
        r = me % 8
        my = r // 2
        mx = (r % 2) ^ (my & 1)

        def logical(px, py, pz):
            return pz * 8 + py * 2 + (px ^ (py & 1))

        flips = [
            (mx, logical(mx ^ 1, my, mz)),
            (my & 1, logical(mx, my ^ 1, mz)),
            (mz & 1, logical(mx, my, mz ^ 1)),
            ((my // 2) & 1, logical(mx, my ^ 2, mz)),
            ((mz // 2) & 1, logical(mx, my, mz ^ 2)),
        ]

        acc_ref[...] = x_ref[...]

        offs = [0]
        off = 0
        for k in range(N_DIMS):
            w2 = half[k]
            bit, partner = flips[k]
            send_off = off + (1 - bit) * w2
            keep_off = off + bit * w2
            stage_ref[pl.ds(0, w2), :] = acc_ref[pl.ds(send_off, w2), :].astype(
                jnp.bfloat16
            )
            rdma = pltpu.make_async_remote_copy(
                src_ref=stage_ref.at[pl.ds(0, w2), :],
                dst_ref=rs_recv[k],
                send_sem=send_sems.at[k],
                recv_sem=recv_sems.at[k],
                device_id=partner,
                device_id_type=pl.DeviceIdType.LOGICAL,
            )
            rdma.start()
            rdma.wait()
            acc_ref[pl.ds(keep_off, w2), :] += rs_recv[k][...].astype(jnp.float32)
            off = keep_off
            offs.append(off)

        for j in range(N_DIMS):
            k = N_DIMS - 1 - j
            w2 = half[k]
            bit, partner = flips[k]
            held_off = offs[k + 1]
            other_off = offs[k] + (1 - bit) * w2
            stage_ref[pl.ds(0, w2), :] = acc_ref[pl.ds(held_off, w2), :].astype(
                jnp.bfloat16
            )
            rdma = pltpu.make_async_remote_copy(
                src_ref=stage_ref.at[pl.ds(0, w2), :],
                dst_ref=ag_recv[j],
                send_sem=send_sems.at[N_DIMS + j],
                recv_sem=recv_sems.at[N_DIMS + j],
                device_id=partner,
                device_id_type=pl.DeviceIdType.LOGICAL,
            )
            rdma.start()
            rdma.wait()
            acc_ref[pl.ds(other_off, w2), :] = ag_recv[j][...].astype(jnp.float32)

        out_ref[...] = acc_ref[...]

    return pl.pallas_call(
        body,
        out_shape=jax.ShapeDtypeStruct((m, n), jnp.float32),
        in_specs=[pl.BlockSpec(memory_space=pltpu.VMEM)],
        out_specs=pl.BlockSpec(memory_space=pltpu.VMEM),
        scratch_shapes=[
            pltpu.VMEM((m, n), jnp.float32),
            pltpu.VMEM((m // 2, n), jnp.bfloat16),
            *[pltpu.VMEM((half[k], n), jnp.bfloat16) for k in range(N_DIMS)],
            *[pltpu.VMEM((half[N_DIMS - 1 - j], n), jnp.bfloat16) for j in range(N_DIMS)],
            pltpu.SemaphoreType.DMA((2 * N_DIMS,)),
            pltpu.SemaphoreType.DMA((2 * N_DIMS,)),
        ],
    )(x)


# device time: 35691 ns/iter; 1.8157x vs baseline; 1.1842x over previous
import jax
import jax.numpy as jnp
from jax import lax
from jax.experimental import pallas as pl
from jax.experimental.pallas import tpu as pltpu

N_DEV = 32
N_DIMS = 5


def kernel(x):
    m, n = x.shape
    half = [m >> (k + 1) for k in range(N_DIMS)]
    stage_off = [0]
    for k in range(1, N_DIMS):
        stage_off.append(stage_off[-1] + half[k - 1])
    stage_rows = stage_off[-1] + half[-1]

    def body(x_ref, out_ref, acc_ref, stage_ref, g_ref, *rest):
        rs_recv = rest[0:N_DIMS]
        send_sems, recv_sems = rest[N_DIMS], rest[N_DIMS + 1]

        me = lax.axis_index("i")
        mz = me // 8
        r = me % 8
        my = r // 2
        mx = (r % 2) ^ (my & 1)

        def logical(px, py, pz):
            return pz * 8 + py * 2 + (px ^ (py & 1))

        flips = [
            (mx, logical(mx ^ 1, my, mz)),
            (my & 1, logical(mx, my ^ 1, mz)),
            (mz & 1, logical(mx, my, mz ^ 1)),
            ((my // 2) & 1, logical(mx, my ^ 2, mz)),
            ((mz // 2) & 1, logical(mx, my, mz ^ 2)),
        ]

        barrier_sem = pltpu.get_barrier_semaphore()
        for _, partner in flips:
            pl.semaphore_signal(
                barrier_sem,
                inc=1,
                device_id=partner,
                device_id_type=pl.DeviceIdType.LOGICAL,
            )
        pl.semaphore_wait(barrier_sem, N_DIMS)

        acc_ref[...] = x_ref[...]

        descs = []
        offs = [0]
        off = 0
        for k in range(N_DIMS):
            w2 = half[k]
            bit, partner = flips[k]
            send_off = off + (1 - bit) * w2
            keep_off = off + bit * w2
            s = stage_off[k]
            stage_ref[pl.ds(s, w2), :] = acc_ref[pl.ds(send_off, w2), :].astype(
                jnp.bfloat16
            )
            rdma = pltpu.make_async_remote_copy(
                src_ref=stage_ref.at[pl.ds(s, w2), :],
                dst_ref=rs_recv[k],
                send_sem=send_sems.at[k],
                recv_sem=recv_sems.at[k],
                device_id=partner,
                device_id_type=pl.DeviceIdType.LOGICAL,
            )
            rdma.start()
            rdma.wait_recv()
            descs.append(rdma)
            val = acc_ref[pl.ds(keep_off, w2), :] + rs_recv[k][...].astype(jnp.float32)
            acc_ref[pl.ds(keep_off, w2), :] = val
            if k == N_DIMS - 1:
                g_ref[pl.ds(keep_off, w2), :] = val.astype(jnp.bfloat16)
            off = keep_off
            offs.append(off)

        for j in range(N_DIMS):
            k = N_DIMS - 1 - j
            w2 = half[k]
            held_off = offs[k + 1]
            _, partner = flips[k]
            rdma = pltpu.make_async_remote_copy(
                src_ref=g_ref.at[pl.ds(held_off, w2), :],
                dst_ref=g_ref.at[pl.ds(held_off, w2), :],
                send_sem=send_sems.at[N_DIMS + j],
                recv_sem=recv_sems.at[N_DIMS + j],
                device_id=partner,
                device_id_type=pl.DeviceIdType.LOGICAL,
            )
            rdma.start()
            rdma.wait_recv()
            descs.append(rdma)

        out_ref[...] = g_ref[...].astype(jnp.float32)

        for rdma in descs:
            rdma.wait_send()

    return pl.pallas_call(
        body,
        out_shape=jax.ShapeDtypeStruct((m, n), jnp.float32),
        in_specs=[pl.BlockSpec(memory_space=pltpu.VMEM)],
        out_specs=pl.BlockSpec(memory_space=pltpu.VMEM),
        scratch_shapes=[
            pltpu.VMEM((m, n), jnp.float32),
            pltpu.VMEM((stage_rows, n), jnp.bfloat16),
            pltpu.VMEM((m, n), jnp.bfloat16),
            *[pltpu.VMEM((half[k], n), jnp.bfloat16) for k in range(N_DIMS)],
            pltpu.SemaphoreType.DMA((2 * N_DIMS,)),
            pltpu.SemaphoreType.DMA((2 * N_DIMS,)),
        ],
        compiler_params=pltpu.CompilerParams(collective_id=0),
    )(x)


# device time: 33209 ns/iter; 1.9514x vs baseline; 1.0747x over previous
import jax
import jax.numpy as jnp
from jax import lax
from jax.experimental import pallas as pl
from jax.experimental.pallas import tpu as pltpu

N_DEV = 32
N_ROUNDS = 8

A_DIMS = ("x", "y1", "z1", "y2", "z2", "z1", "y1", "x")
B_DIMS = ("z1", "x", "y1", "z2", "y2", "y1", "x", "z1")


def kernel(x):
    m, n = x.shape
    n2 = n
    rs_sz = [m >> 1, m >> 2, m >> 3]
    w = m >> 3
    stg = [0, 256, 384, 448, 512]
    stg_rows = 576

    def body(x_ref, out_ref, acc_ref, stage_ref, recv_ref, g_ref, send_sems, recv_sems):
        me = lax.axis_index("i")
        mz = me // 8
        r8 = me % 8
        my = r8 // 2
        mx = (r8 % 2) ^ (my & 1)

        def logical(px, py, pz):
            return pz * 8 + py * 2 + (px ^ (py & 1))

        def dim_info(d):
            if d == "x":
                return mx, logical(mx ^ 1, my, mz)
            if d == "y1":
                return my & 1, logical(mx, my ^ 1, mz)
            if d == "z1":
                return mz & 1, logical(mx, my, mz ^ 1)
            if d == "y2":
                return (my // 2) & 1, logical(mx, my ^ 2, mz)
            return (mz // 2) & 1, logical(mx, my, mz ^ 2)

        barrier_sem = pltpu.get_barrier_semaphore()
        for d in ("x", "y1", "z1", "y2", "z2"):
            _, partner = dim_info(d)
            pl.semaphore_signal(
                barrier_sem,
                inc=1,
                device_id=partner,
                device_id_type=pl.DeviceIdType.LOGICAL,
            )
        pl.semaphore_wait(barrier_sem, 5)

        acc_ref[...] = x_ref[...]

        chains = [
            {"dims": A_DIMS, "c0": 0, "sb": 0, "off": 0, "offs": [0]},
        ]
        descs = []

        for r in range(N_ROUNDS):
            rdmas = []
            for ch in chains:
                bit, partner = dim_info(ch["dims"][r])
                c0 = ch["c0"]
                si = ch["sb"] + r
                if r < 3:
                    w2 = rs_sz[r]
                    send_off = ch["off"] + (1 - bit) * w2
                    src = acc_ref[pl.ds(send_off, w2), pl.ds(c0, n2)]
                elif r < 5:
                    w2 = w
                    src = acc_ref[pl.ds(ch["off"], w2), pl.ds(c0, n2)]
                else:
                    k = N_ROUNDS - 1 - r
                    held_off = ch["offs"][k + 1]
                    w2 = rs_sz[k]
                    rdma = pltpu.make_async_remote_copy(
                        src_ref=g_ref.at[pl.ds(held_off, w2), pl.ds(c0, n2)],
                        dst_ref=g_ref.at[pl.ds(held_off, w2), pl.ds(c0, n2)],
                        send_sem=send_sems.at[si],
                        recv_sem=recv_sems.at[si],
                        device_id=partner,
                        device_id_type=pl.DeviceIdType.LOGICAL,
                    )
                    rdma.start()
                    rdmas.append(rdma)
                    continue
                so = stg[r]
                stage_ref[pl.ds(so, w2), pl.ds(c0, n2)] = src.astype(jnp.bfloat16)
                rdma = pltpu.make_async_remote_copy(
                    src_ref=stage_ref.at[pl.ds(so, w2), pl.ds(c0, n2)],
                    dst_ref=recv_ref.at[pl.ds(so, w2), pl.ds(c0, n2)],
                    send_sem=send_sems.at[si],
                    recv_sem=recv_sems.at[si],
                    device_id=partner,
                    device_id_type=pl.DeviceIdType.LOGICAL,
                )
                rdma.start()
                rdmas.append(rdma)

            for ch, rdma in zip(chains, rdmas):
                rdma.wait_recv()
                descs.append(rdma)
                bit, _ = dim_info(ch["dims"][r])
                c0 = ch["c0"]
                if r < 3:
                    w2 = rs_sz[r]
                    keep_off = ch["off"] + bit * w2
                    so = stg[r]
                    val = acc_ref[pl.ds(keep_off, w2), pl.ds(c0, n2)] + recv_ref[
                        pl.ds(so, w2), pl.ds(c0, n2)
                    ].astype(jnp.float32)
                    acc_ref[pl.ds(keep_off, w2), pl.ds(c0, n2)] = val
                    ch["off"] = keep_off
                    ch["offs"].append(keep_off)
                elif r < 5:
                    so = stg[r]
                    val = acc_ref[pl.ds(ch["off"], w), pl.ds(c0, n2)] + recv_ref[
                        pl.ds(so, w), pl.ds(c0, n2)
                    ].astype(jnp.float32)
                    acc_ref[pl.ds(ch["off"], w), pl.ds(c0, n2)] = val
                    if r == 4:
                        g_ref[pl.ds(ch["off"], w), pl.ds(c0, n2)] = val.astype(
                            jnp.bfloat16
                        )

        out_ref[...] = g_ref[...].astype(jnp.float32)

        for rdma in descs:
            rdma.wait_send()

    return pl.pallas_call(
        body,
        out_shape=jax.ShapeDtypeStruct((m, n), jnp.float32),
        in_specs=[pl.BlockSpec(memory_space=pltpu.VMEM)],
        out_specs=pl.BlockSpec(memory_space=pltpu.VMEM),
        scratch_shapes=[
            pltpu.VMEM((m, n), jnp.float32),
            pltpu.VMEM((stg_rows, n), jnp.bfloat16),
            pltpu.VMEM((stg_rows, n), jnp.bfloat16),
            pltpu.VMEM((m, n), jnp.bfloat16),
            pltpu.SemaphoreType.DMA((2 * N_ROUNDS,)),
            pltpu.SemaphoreType.DMA((2 * N_ROUNDS,)),
        ],
        compiler_params=pltpu.CompilerParams(collective_id=0),
    )(x)


# device time: 29092 ns/iter; 2.2276x vs baseline; 1.1415x over previous
import jax
import jax.numpy as jnp
from jax import lax
from jax.experimental import pallas as pl
from jax.experimental.pallas import tpu as pltpu

N_DEV = 32
N_ROUNDS = 8

A_DIMS = ("x", "y1", "z1", "y2", "z2", "z1", "y1", "x")
B_DIMS = ("z1", "x", "y1", "z2", "y2", "y1", "x", "z1")


def kernel(x):
    m, n = x.shape
    n2 = n // 2
    rs_sz = [m >> 1, m >> 2, m >> 3]
    w = m >> 3
    stg = [0, 256, 384, 448, 512]
    stg_rows = 576

    def body(
        x_ref,
        out_ref,
        acc_ref,
        stage_a,
        recv_a,
        g_a,
        stage_b,
        recv_b,
        g_b,
        send_sems,
        recv_sems,
    ):
        me = lax.axis_index("i")
        mz = me // 8
        r8 = me % 8
        my = r8 // 2
        mx = (r8 % 2) ^ (my & 1)

        def logical(px, py, pz):
            return pz * 8 + py * 2 + (px ^ (py & 1))

        def dim_info(d):
            if d == "x":
                return mx, logical(mx ^ 1, my, mz)
            if d == "y1":
                return my & 1, logical(mx, my ^ 1, mz)
            if d == "z1":
                return mz & 1, logical(mx, my, mz ^ 1)
            if d == "y2":
                return (my // 2) & 1, logical(mx, my ^ 2, mz)
            return (mz // 2) & 1, logical(mx, my, mz ^ 2)

        barrier_sem = pltpu.get_barrier_semaphore()
        for d in ("x", "y1", "z1", "y2", "z2"):
            _, partner = dim_info(d)
            pl.semaphore_signal(
                barrier_sem,
                inc=1,
                device_id=partner,
                device_id_type=pl.DeviceIdType.LOGICAL,
            )
        pl.semaphore_wait(barrier_sem, 5)

        acc_ref[...] = x_ref[...]

        chains = [
            {
                "dims": A_DIMS,
                "c0": 0,
                "sb": 0,
                "off": 0,
                "offs": [0],
                "stage": stage_a,
                "recv": recv_a,
                "g": g_a,
            },
            {
                "dims": B_DIMS,
                "c0": n2,
                "sb": N_ROUNDS,
                "off": 0,
                "offs": [0],
                "stage": stage_b,
                "recv": recv_b,
                "g": g_b,
            },
        ]
        descs = []

        for r in range(N_ROUNDS):
            rdmas = []
            for ch in chains:
                bit, partner = dim_info(ch["dims"][r])
                c0 = ch["c0"]
                si = ch["sb"] + r
                if r < 3:
                    w2 = rs_sz[r]
                    send_off = ch["off"] + (1 - bit) * w2
                    src = acc_ref[pl.ds(send_off, w2), pl.ds(c0, n2)]
                elif r < 5:
                    w2 = w
                    src = acc_ref[pl.ds(ch["off"], w2), pl.ds(c0, n2)]
                else:
                    k = N_ROUNDS - 1 - r
                    held_off = ch["offs"][k + 1]
                    w2 = rs_sz[k]
                    rdma = pltpu.make_async_remote_copy(
                        src_ref=ch["g"].at[pl.ds(held_off, w2), :],
                        dst_ref=ch["g"].at[pl.ds(held_off, w2), :],
                        send_sem=send_sems.at[si],
                        recv_sem=recv_sems.at[si],
                        device_id=partner,
                        device_id_type=pl.DeviceIdType.LOGICAL,
                    )
                    rdma.start()
                    rdmas.append(rdma)
                    continue
                so = stg[r]
                ch["stage"][pl.ds(so, w2), :] = src.astype(jnp.bfloat16)
                rdma = pltpu.make_async_remote_copy(
                    src_ref=ch["stage"].at[pl.ds(so, w2), :],
                    dst_ref=ch["recv"].at[pl.ds(so, w2), :],
                    send_sem=send_sems.at[si],
                    recv_sem=recv_sems.at[si],
                    device_id=partner,
                    device_id_type=pl.DeviceIdType.LOGICAL,
                )
                rdma.start()
                rdmas.append(rdma)

            for ch, rdma in zip(chains, rdmas):
                rdma.wait_recv()
                descs.append(rdma)
                bit, _ = dim_info(ch["dims"][r])
                c0 = ch["c0"]
                if r < 3:
                    w2 = rs_sz[r]
                    keep_off = ch["off"] + bit * w2
                    so = stg[r]
                    val = acc_ref[pl.ds(keep_off, w2), pl.ds(c0, n2)] + ch["recv"][
                        pl.ds(so, w2), :
                    ].astype(jnp.float32)
                    acc_ref[pl.ds(keep_off, w2), pl.ds(c0, n2)] = val
                    ch["off"] = keep_off
                    ch["offs"].append(keep_off)
                elif r < 5:
                    so = stg[r]
                    val = acc_ref[pl.ds(ch["off"], w), pl.ds(c0, n2)] + ch["recv"][
                        pl.ds(so, w), :
                    ].astype(jnp.float32)
                    acc_ref[pl.ds(ch["off"], w), pl.ds(c0, n2)] = val
                    if r == 4:
                        ch["g"][pl.ds(ch["off"], w), :] = val.astype(jnp.bfloat16)

        out_ref[:, pl.ds(0, n2)] = g_a[...].astype(jnp.float32)
        out_ref[:, pl.ds(n2, n2)] = g_b[...].astype(jnp.float32)

        for rdma in descs:
            rdma.wait_send()

    return pl.pallas_call(
        body,
        out_shape=jax.ShapeDtypeStruct((m, n), jnp.float32),
        in_specs=[pl.BlockSpec(memory_space=pltpu.VMEM)],
        out_specs=pl.BlockSpec(memory_space=pltpu.VMEM),
        scratch_shapes=[
            pltpu.VMEM((m, n), jnp.float32),
            pltpu.VMEM((stg_rows, n2), jnp.bfloat16),
            pltpu.VMEM((stg_rows, n2), jnp.bfloat16),
            pltpu.VMEM((m, n2), jnp.bfloat16),
            pltpu.VMEM((stg_rows, n2), jnp.bfloat16),
            pltpu.VMEM((stg_rows, n2), jnp.bfloat16),
            pltpu.VMEM((m, n2), jnp.bfloat16),
            pltpu.SemaphoreType.DMA((2 * N_ROUNDS,)),
            pltpu.SemaphoreType.DMA((2 * N_ROUNDS,)),
        ],
        compiler_params=pltpu.CompilerParams(collective_id=0),
    )(x)


# device time: 29033 ns/iter; 2.2321x vs baseline; 1.0020x over previous
import jax
import jax.numpy as jnp
from jax import lax
from jax.experimental import pallas as pl
from jax.experimental.pallas import tpu as pltpu

N_DEV = 32
N_ROUNDS = 8

A_DIMS = ("x", "y1", "z1", "y2", "z2", "z1", "y1", "x")
B_DIMS = ("z1", "x", "y1", "z2", "y2", "y1", "x", "z1")


def kernel(x):
    m, n = x.shape
    n2 = n // 2
    rs_sz = [m >> 1, m >> 2, m >> 3]
    w = m >> 3
    stg = [0, 256, 384, 448, 512]
    stg_rows = 576

    def body(
        x_ref,
        out_ref,
        acc_ref,
        stage_a,
        recv_a,
        g_a,
        stage_b,
        recv_b,
        g_b,
        send_sems,
        recv_sems,
    ):
        me = lax.axis_index("i")
        mz = me // 8
        r8 = me % 8
        my = r8 // 2
        mx = (r8 % 2) ^ (my & 1)

        def logical(px, py, pz):
            return pz * 8 + py * 2 + (px ^ (py & 1))

        def dim_info(d):
            if d == "x":
                return mx, logical(mx ^ 1, my, mz)
            if d == "y1":
                return my & 1, logical(mx, my ^ 1, mz)
            if d == "z1":
                return mz & 1, logical(mx, my, mz ^ 1)
            if d == "y2":
                return (my // 2) & 1, logical(mx, my ^ 2, mz)
            return (mz // 2) & 1, logical(mx, my, mz ^ 2)

        barrier_sem = pltpu.get_barrier_semaphore()
        for d in ("x", "y1", "z1", "y2", "z2"):
            _, partner = dim_info(d)
            pl.semaphore_signal(
                barrier_sem,
                inc=1,
                device_id=partner,
                device_id_type=pl.DeviceIdType.LOGICAL,
            )
        pl.semaphore_wait(barrier_sem, 5)

        acc_ref[...] = x_ref[...]

        chains = [
            {
                "dims": A_DIMS,
                "c0": 0,
                "sb": 0,
                "off": 0,
                "offs": [0],
                "stage": stage_a,
                "recv": recv_a,
                "g": g_a,
            },
            {
                "dims": B_DIMS,
                "c0": n2,
                "sb": N_ROUNDS,
                "off": 0,
                "offs": [0],
                "stage": stage_b,
                "recv": recv_b,
                "g": g_b,
            },
        ]
        descs = []

        def issue(ch, r):
            bit, partner = dim_info(ch["dims"][r])
            c0 = ch["c0"]
            si = ch["sb"] + r
            if r < 3:
                w2 = rs_sz[r]
                send_off = ch["off"] + (1 - bit) * w2
                src = acc_ref[pl.ds(send_off, w2), pl.ds(c0, n2)]
            elif r < 5:
                w2 = w
                src = acc_ref[pl.ds(ch["off"], w2), pl.ds(c0, n2)]
            else:
                k = N_ROUNDS - 1 - r
                held_off = ch["offs"][k + 1]
                w2 = rs_sz[k]
                rdma = pltpu.make_async_remote_copy(
                    src_ref=ch["g"].at[pl.ds(held_off, w2), :],
                    dst_ref=ch["g"].at[pl.ds(held_off, w2), :],
                    send_sem=send_sems.at[si],
                    recv_sem=recv_sems.at[si],
                    device_id=partner,
                    device_id_type=pl.DeviceIdType.LOGICAL,
                )
                rdma.start()
                return rdma
            so = stg[r]
            ch["stage"][pl.ds(so, w2), :] = src.astype(jnp.bfloat16)
            rdma = pltpu.make_async_remote_copy(
                src_ref=ch["stage"].at[pl.ds(so, w2), :],
                dst_ref=ch["recv"].at[pl.ds(so, w2), :],
                send_sem=send_sems.at[si],
                recv_sem=recv_sems.at[si],
                device_id=partner,
                device_id_type=pl.DeviceIdType.LOGICAL,
            )
            rdma.start()
            return rdma

        def process(ch, r, rdma):
            rdma.wait_recv()
            descs.append(rdma)
            bit, _ = dim_info(ch["dims"][r])
            c0 = ch["c0"]
            if r < 3:
                w2 = rs_sz[r]
                keep_off = ch["off"] + bit * w2
                so = stg[r]
                val = acc_ref[pl.ds(keep_off, w2), pl.ds(c0, n2)] + ch["recv"][
                    pl.ds(so, w2), :
                ].astype(jnp.float32)
                acc_ref[pl.ds(keep_off, w2), pl.ds(c0, n2)] = val
                ch["off"] = keep_off
                ch["offs"].append(keep_off)
            elif r < 5:
                so = stg[r]
                val = acc_ref[pl.ds(ch["off"], w), pl.ds(c0, n2)] + ch["recv"][
                    pl.ds(so, w), :
                ].astype(jnp.float32)
                acc_ref[pl.ds(ch["off"], w), pl.ds(c0, n2)] = val
                if r == 4:
                    ch["g"][pl.ds(ch["off"], w), :] = val.astype(jnp.bfloat16)

        pending = [issue(ch, 0) for ch in chains]
        for r in range(N_ROUNDS):
            for ci, ch in enumerate(chains):
                process(ch, r, pending[ci])
                if r + 1 < N_ROUNDS:
                    pending[ci] = issue(ch, r + 1)

        out_ref[:, pl.ds(0, n2)] = g_a[...].astype(jnp.float32)
        out_ref[:, pl.ds(n2, n2)] = g_b[...].astype(jnp.float32)

        for rdma in descs:
            rdma.wait_send()

    return pl.pallas_call(
        body,
        out_shape=jax.ShapeDtypeStruct((m, n), jnp.float32),
        in_specs=[pl.BlockSpec(memory_space=pltpu.VMEM)],
        out_specs=pl.BlockSpec(memory_space=pltpu.VMEM),
        scratch_shapes=[
            pltpu.VMEM((m, n), jnp.float32),
            pltpu.VMEM((stg_rows, n2), jnp.bfloat16),
            pltpu.VMEM((stg_rows, n2), jnp.bfloat16),
            pltpu.VMEM((m, n2), jnp.bfloat16),
            pltpu.VMEM((stg_rows, n2), jnp.bfloat16),
            pltpu.VMEM((stg_rows, n2), jnp.bfloat16),
            pltpu.VMEM((m, n2), jnp.bfloat16),
            pltpu.SemaphoreType.DMA((2 * N_ROUNDS,)),
            pltpu.SemaphoreType.DMA((2 * N_ROUNDS,)),
        ],
        compiler_params=pltpu.CompilerParams(collective_id=0),
    )(x)


# device time: 26872 ns/iter; 2.4116x vs baseline; 1.0804x over previous
import jax
import jax.numpy as jnp
from jax import lax
from jax.experimental import pallas as pl
from jax.experimental.pallas import tpu as pltpu

N_DEV = 32
N_ROUNDS = 8

A_DIMS = ("x", "y1", "z1", "y2", "z2", "z1", "y1", "x")
B_DIMS = ("z1", "x", "y1", "z2", "y2", "y1", "x", "z1")


N_CHAINS = 4


def kernel(x):
    m, n = x.shape
    n2 = n // N_CHAINS
    rs_sz = [m >> 1, m >> 2, m >> 3]
    w = m >> 3
    stg = [0, 256, 384, 448, 512]
    stg_rows = 576

    def body(x_ref, out_ref, acc_ref, *rest):
        bufs = rest[: 3 * N_CHAINS]
        send_sems, recv_sems = rest[3 * N_CHAINS], rest[3 * N_CHAINS + 1]
        me = lax.axis_index("i")
        mz = me // 8
        r8 = me % 8
        my = r8 // 2
        mx = (r8 % 2) ^ (my & 1)

        def logical(px, py, pz):
            return pz * 8 + py * 2 + (px ^ (py & 1))

        def dim_info(d):
            if d == "x":
                return mx, logical(mx ^ 1, my, mz)
            if d == "y1":
                return my & 1, logical(mx, my ^ 1, mz)
            if d == "z1":
                return mz & 1, logical(mx, my, mz ^ 1)
            if d == "y2":
                return (my // 2) & 1, logical(mx, my ^ 2, mz)
            return (mz // 2) & 1, logical(mx, my, mz ^ 2)

        barrier_sem = pltpu.get_barrier_semaphore()
        for d in ("x", "y1", "z1", "y2", "z2"):
            _, partner = dim_info(d)
            pl.semaphore_signal(
                barrier_sem,
                inc=1,
                device_id=partner,
                device_id_type=pl.DeviceIdType.LOGICAL,
            )
        pl.semaphore_wait(barrier_sem, 5)

        acc_ref[...] = x_ref[...]

        chains = [
            {
                "dims": A_DIMS if ci % 2 == 0 else B_DIMS,
                "c0": ci * n2,
                "sb": ci * N_ROUNDS,
                "off": 0,
                "offs": [0],
                "stage": bufs[3 * ci],
                "recv": bufs[3 * ci + 1],
                "g": bufs[3 * ci + 2],
            }
            for ci in range(N_CHAINS)
        ]
        descs = []

        def issue(ch, r):
            bit, partner = dim_info(ch["dims"][r])
            c0 = ch["c0"]
            si = ch["sb"] + r
            if r < 3:
                w2 = rs_sz[r]
                send_off = ch["off"] + (1 - bit) * w2
                src = acc_ref[pl.ds(send_off, w2), pl.ds(c0, n2)]
            elif r < 5:
                w2 = w
                src = acc_ref[pl.ds(ch["off"], w2), pl.ds(c0, n2)]
            else:
                k = N_ROUNDS - 1 - r
                held_off = ch["offs"][k + 1]
                w2 = rs_sz[k]
                rdma = pltpu.make_async_remote_copy(
                    src_ref=ch["g"].at[pl.ds(held_off, w2), :],
                    dst_ref=ch["g"].at[pl.ds(held_off, w2), :],
                    send_sem=send_sems.at[si],
                    recv_sem=recv_sems.at[si],
                    device_id=partner,
                    device_id_type=pl.DeviceIdType.LOGICAL,
                )
                rdma.start()
                return rdma
            so = stg[r]
            ch["stage"][pl.ds(so, w2), :] = src.astype(jnp.bfloat16)
            rdma = pltpu.make_async_remote_copy(
                src_ref=ch["stage"].at[pl.ds(so, w2), :],
                dst_ref=ch["recv"].at[pl.ds(so, w2), :],
                send_sem=send_sems.at[si],
                recv_sem=recv_sems.at[si],
                device_id=partner,
                device_id_type=pl.DeviceIdType.LOGICAL,
            )
            rdma.start()
            return rdma

        def process(ch, r, rdma):
            rdma.wait_recv()
            descs.append(rdma)
            bit, _ = dim_info(ch["dims"][r])
            c0 = ch["c0"]
            if r < 3:
                w2 = rs_sz[r]
                keep_off = ch["off"] + bit * w2
                so = stg[r]
                val = acc_ref[pl.ds(keep_off, w2), pl.ds(c0, n2)] + ch["recv"][
                    pl.ds(so, w2), :
                ].astype(jnp.float32)
                acc_ref[pl.ds(keep_off, w2), pl.ds(c0, n2)] = val
                ch["off"] = keep_off
                ch["offs"].append(keep_off)
            elif r < 5:
                so = stg[r]
                val = acc_ref[pl.ds(ch["off"], w), pl.ds(c0, n2)] + ch["recv"][
                    pl.ds(so, w), :
                ].astype(jnp.float32)
                acc_ref[pl.ds(ch["off"], w), pl.ds(c0, n2)] = val
                if r == 4:
                    ch["g"][pl.ds(ch["off"], w), :] = val.astype(jnp.bfloat16)

        pending = [issue(ch, 0) for ch in chains]
        for r in range(N_ROUNDS):
            for ci, ch in enumerate(chains):
                process(ch, r, pending[ci])
                if r + 1 < N_ROUNDS:
                    pending[ci] = issue(ch, r + 1)

        for ch in chains:
            out_ref[:, pl.ds(ch["c0"], n2)] = ch["g"][...].astype(jnp.float32)

        for rdma in descs:
            rdma.wait_send()

    return pl.pallas_call(
        body,
        out_shape=jax.ShapeDtypeStruct((m, n), jnp.float32),
        in_specs=[pl.BlockSpec(memory_space=pltpu.VMEM)],
        out_specs=pl.BlockSpec(memory_space=pltpu.VMEM),
        scratch_shapes=[
            pltpu.VMEM((m, n), jnp.float32),
            *[
                shape
                for _ in range(N_CHAINS)
                for shape in (
                    pltpu.VMEM((stg_rows, n2), jnp.bfloat16),
                    pltpu.VMEM((stg_rows, n2), jnp.bfloat16),
                    pltpu.VMEM((m, n2), jnp.bfloat16),
                )
            ],
            pltpu.SemaphoreType.DMA((N_CHAINS * N_ROUNDS,)),
            pltpu.SemaphoreType.DMA((N_CHAINS * N_ROUNDS,)),
        ],
        compiler_params=pltpu.CompilerParams(collective_id=0),
    )(x)


# device time: 26769 ns/iter; 2.4209x vs baseline; 1.0038x over previous
import jax
import jax.numpy as jnp
from jax import lax
from jax.experimental import pallas as pl
from jax.experimental.pallas import tpu as pltpu

N_DEV = 32
N_ROUNDS = 8

A_DIMS = ("x", "y1", "z1", "y2", "z2", "z1", "y1", "x")
B_DIMS = ("z1", "x", "y1", "z2", "y2", "y1", "x", "z1")


N_CHAINS = 4


def kernel(x):
    m, n = x.shape
    n2 = n // N_CHAINS
    rs_sz = [m >> 1, m >> 2, m >> 3]
    w = m >> 3
    stg = [0, 256, 384, 448, 512]
    stg_rows = 576

    def body(x_ref, out_ref, acc_ref, *rest):
        bufs = rest[: 3 * N_CHAINS]
        send_sems, recv_sems = rest[3 * N_CHAINS], rest[3 * N_CHAINS + 1]
        me = lax.axis_index("i")
        mz = me // 8
        r8 = me % 8
        my = r8 // 2
        mx = (r8 % 2) ^ (my & 1)

        def logical(px, py, pz):
            return pz * 8 + py * 2 + (px ^ (py & 1))

        def dim_info(d):
            if d == "x":
                return mx, logical(mx ^ 1, my, mz)
            if d == "y1":
                return my & 1, logical(mx, my ^ 1, mz)
            if d == "z1":
                return mz & 1, logical(mx, my, mz ^ 1)
            if d == "y2":
                return (my // 2) & 1, logical(mx, my ^ 2, mz)
            return (mz // 2) & 1, logical(mx, my, mz ^ 2)

        chains = [
            {
                "dims": A_DIMS if ci % 2 == 0 else B_DIMS,
                "c0": ci * n2,
                "sb": ci * N_ROUNDS,
                "off": 0,
                "offs": [0],
                "stage": bufs[3 * ci],
                "recv": bufs[3 * ci + 1],
                "g": bufs[3 * ci + 2],
            }
            for ci in range(N_CHAINS)
        ]
        descs = []

        for ch in chains:
            bit, _ = dim_info(ch["dims"][0])
            w2 = rs_sz[0]
            send_off = (1 - bit) * w2
            ch["stage"][pl.ds(0, w2), :] = x_ref[
                pl.ds(send_off, w2), pl.ds(ch["c0"], n2)
            ].astype(jnp.bfloat16)

        barrier_sem = pltpu.get_barrier_semaphore()
        for d in ("x", "y1", "z1", "y2", "z2"):
            _, partner = dim_info(d)
            pl.semaphore_signal(
                barrier_sem,
                inc=1,
                device_id=partner,
                device_id_type=pl.DeviceIdType.LOGICAL,
            )
        pl.semaphore_wait(barrier_sem, 5)

        def issue(ch, r):
            bit, partner = dim_info(ch["dims"][r])
            c0 = ch["c0"]
            si = ch["sb"] + r
            if r == 0:
                w2 = rs_sz[0]
                rdma = pltpu.make_async_remote_copy(
                    src_ref=ch["stage"].at[pl.ds(0, w2), :],
                    dst_ref=ch["recv"].at[pl.ds(0, w2), :],
                    send_sem=send_sems.at[si],
                    recv_sem=recv_sems.at[si],
                    device_id=partner,
                    device_id_type=pl.DeviceIdType.LOGICAL,
                )
                rdma.start()
                return rdma
            if r < 3:
                w2 = rs_sz[r]
                send_off = ch["off"] + (1 - bit) * w2
                src = acc_ref[pl.ds(send_off, w2), pl.ds(c0, n2)]
            elif r < 5:
                w2 = w
                src = acc_ref[pl.ds(ch["off"], w2), pl.ds(c0, n2)]
            else:
                k = N_ROUNDS - 1 - r
                held_off = ch["offs"][k + 1]
                w2 = rs_sz[k]
                rdma = pltpu.make_async_remote_copy(
                    src_ref=ch["g"].at[pl.ds(held_off, w2), :],
                    dst_ref=ch["g"].at[pl.ds(held_off, w2), :],
                    send_sem=send_sems.at[si],
                    recv_sem=recv_sems.at[si],
                    device_id=partner,
                    device_id_type=pl.DeviceIdType.LOGICAL,
                )
                rdma.start()
                return rdma
            so = stg[r]
            ch["stage"][pl.ds(so, w2), :] = src.astype(jnp.bfloat16)
            rdma = pltpu.make_async_remote_copy(
                src_ref=ch["stage"].at[pl.ds(so, w2), :],
                dst_ref=ch["recv"].at[pl.ds(so, w2), :],
                send_sem=send_sems.at[si],
                recv_sem=recv_sems.at[si],
                device_id=partner,
                device_id_type=pl.DeviceIdType.LOGICAL,
            )
            rdma.start()
            return rdma

        def process(ch, r, rdma):
            rdma.wait_recv()
            descs.append(rdma)
            bit, _ = dim_info(ch["dims"][r])
            c0 = ch["c0"]
            if r < 3:
                w2 = rs_sz[r]
                keep_off = ch["off"] + bit * w2
                so = stg[r]
                lhs_ref = x_ref if r == 0 else acc_ref
                val = lhs_ref[pl.ds(keep_off, w2), pl.ds(c0, n2)] + ch["recv"][
                    pl.ds(so, w2), :
                ].astype(jnp.float32)
                acc_ref[pl.ds(keep_off, w2), pl.ds(c0, n2)] = val
                ch["off"] = keep_off
                ch["offs"].append(keep_off)
            elif r < 5:
                so = stg[r]
                val = acc_ref[pl.ds(ch["off"], w), pl.ds(c0, n2)] + ch["recv"][
                    pl.ds(so, w), :
                ].astype(jnp.float32)
                acc_ref[pl.ds(ch["off"], w), pl.ds(c0, n2)] = val
                if r == 4:
                    ch["g"][pl.ds(ch["off"], w), :] = val.astype(jnp.bfloat16)
            elif r == N_ROUNDS - 2:
                held = ch["offs"][1]
                out_ref[pl.ds(held, rs_sz[0]), pl.ds(c0, n2)] = ch["g"][
                    pl.ds(held, rs_sz[0]), :
                ].astype(jnp.float32)
            elif r == N_ROUNDS - 1:
                other = ch["offs"][0] + rs_sz[0] - ch["offs"][1]
                out_ref[pl.ds(other, rs_sz[0]), pl.ds(c0, n2)] = ch["g"][
                    pl.ds(other, rs_sz[0]), :
                ].astype(jnp.float32)

        pending = [issue(ch, 0) for ch in chains]
        for r in range(N_ROUNDS):
            for ci, ch in enumerate(chains):
                process(ch, r, pending[ci])
                if r + 1 < N_ROUNDS:
                    pending[ci] = issue(ch, r + 1)

        for rdma in descs:
            rdma.wait_send()

    return pl.pallas_call(
        body,
        out_shape=jax.ShapeDtypeStruct((m, n), jnp.float32),
        in_specs=[pl.BlockSpec(memory_space=pltpu.VMEM)],
        out_specs=pl.BlockSpec(memory_space=pltpu.VMEM),
        scratch_shapes=[
            pltpu.VMEM((m, n), jnp.float32),
            *[
                shape
                for _ in range(N_CHAINS)
                for shape in (
                    pltpu.VMEM((stg_rows, n2), jnp.bfloat16),
                    pltpu.VMEM((stg_rows, n2), jnp.bfloat16),
                    pltpu.VMEM((m, n2), jnp.bfloat16),
                )
            ],
            pltpu.SemaphoreType.DMA((N_CHAINS * N_ROUNDS,)),
            pltpu.SemaphoreType.DMA((N_CHAINS * N_ROUNDS,)),
        ],
        compiler_params=pltpu.CompilerParams(collective_id=0),
    )(x)
